# baseline (speedup 1.0000x reference)
import functools

import jax
import jax.numpy as jnp
from jax import lax
from jax.experimental import pallas as pl
from jax.experimental.pallas import tpu as pltpu
from jax.experimental.pallas import tpu_sc as plsc

B = 1024
HIDDEN = 4096
DEPTH = 8
EPS = 1e-5

BM = 256
BK = 512
BM2 = 64
N_TILES = 32
RPT = B // N_TILES


def _k1_body(h_ref, w_ref, bp_ref, g_ref, be_ref, wg_ref, bg_ref, ptr_ref,
             top_ref, nptr_ref, wslot_ref, sidx_ref, acc_ref, gacc_ref):
    k = pl.program_id(1)
    nk = pl.num_programs(1)

    h_f32 = h_ref[...]
    h_bf = h_f32.astype(jnp.bfloat16)
    w_bf = w_ref[...].astype(jnp.bfloat16)
    part = jnp.dot(h_bf, w_bf, preferred_element_type=jnp.float32)
    wg_bf = wg_ref[...].astype(jnp.bfloat16)
    gpart = jnp.dot(h_bf, wg_bf, preferred_element_type=jnp.float32)

    @pl.when(k == 0)
    def _():
        acc_ref[...] = part
        gacc_ref[...] = gpart

    @pl.when(k != 0)
    def _():
        acc_ref[...] += part
        gacc_ref[...] += gpart

    @pl.when(k == nk - 1)
    def _():
        push = acc_ref[...] + bp_ref[...]
        mean = jnp.mean(push, axis=1, keepdims=True)
        cent = push - mean
        var = jnp.mean(cent * cent, axis=1, keepdims=True)
        normed = cent / jnp.sqrt(var + EPS) * g_ref[...] + be_ref[...]

        logit = gacc_ref[...] + bg_ref[...]
        is_pop = logit > 0.0
        ptr = ptr_ref[...].astype(jnp.int32)
        can_pop = is_pop & (ptr > 0)
        can_push = jnp.logical_not(is_pop) & (ptr < DEPTH)

        m = pl.program_id(0)
        row = m * BM + lax.broadcasted_iota(jnp.int32, (BM, 1), 0)

        top_ref[...] = jnp.where(can_push, normed, 0.0)
        nptr_ref[...] = jnp.where(
            can_pop, ptr - 1, jnp.where(can_push, ptr + 1, ptr)
        ).astype(jnp.float32)
        wslot_ref[...] = jnp.where(can_push, ptr, -1)
        gidx = DEPTH * row + jnp.clip(ptr - 1, 0, DEPTH - 1)
        sidx_ref[...] = jnp.where(jnp.logical_not(can_push) & (ptr > 0),
                                  gidx, -1)


def _k1(hidden_state, w_push, b_push, ln_gamma, ln_beta, w_gate, b_gate,
        pointer):
    nk = HIDDEN // BK
    grid = (B // BM, nk)
    return pl.pallas_call(
        _k1_body,
        grid=grid,
        in_specs=[
            pl.BlockSpec((BM, BK), lambda m, k: (m, k)),
            pl.BlockSpec((BK, HIDDEN), lambda m, k: (k, 0)),
            pl.BlockSpec((1, HIDDEN), lambda m, k: (0, 0)),
            pl.BlockSpec((1, HIDDEN), lambda m, k: (0, 0)),
            pl.BlockSpec((1, HIDDEN), lambda m, k: (0, 0)),
            pl.BlockSpec((BK, 1), lambda m, k: (k, 0)),
            pl.BlockSpec((1, 1), lambda m, k: (0, 0)),
            pl.BlockSpec((BM, 1), lambda m, k: (m, 0)),
        ],
        out_specs=[
            pl.BlockSpec((BM, HIDDEN), lambda m, k: (m, 0)),
            pl.BlockSpec((BM, 1), lambda m, k: (m, 0)),
            pl.BlockSpec((BM, 1), lambda m, k: (m, 0)),
            pl.BlockSpec((BM, 1), lambda m, k: (m, 0)),
        ],
        out_shape=[
            jax.ShapeDtypeStruct((B, HIDDEN), jnp.float32),
            jax.ShapeDtypeStruct((B, 1), jnp.float32),
            jax.ShapeDtypeStruct((B, 1), jnp.int32),
            jax.ShapeDtypeStruct((B, 1), jnp.int32),
        ],
        scratch_shapes=[
            pltpu.VMEM((BM, HIDDEN), jnp.float32),
            pltpu.VMEM((BM, 1), jnp.float32),
        ],
        compiler_params=pltpu.CompilerParams(
            dimension_semantics=("parallel", "arbitrary"),
        ),
    )(hidden_state, w_push, b_push, ln_gamma, ln_beta, w_gate, b_gate,
      pointer)


def _k2_body(stack_ref, top_ref, wslot_ref, out_ref):
    ws = wslot_ref[...].reshape(BM2, 1, 1)
    slot = lax.broadcasted_iota(jnp.int32, (BM2, DEPTH, 1), 1)
    push3 = top_ref[...].reshape(BM2, 1, HIDDEN)
    out_ref[...] = jnp.where(slot == ws, push3, stack_ref[...])


def _k2(stack, top_base, wslot):
    grid = (B // BM2,)
    return pl.pallas_call(
        _k2_body,
        grid=grid,
        in_specs=[
            pl.BlockSpec((BM2, DEPTH, HIDDEN), lambda i: (i, 0, 0)),
            pl.BlockSpec((BM2, HIDDEN), lambda i: (i, 0)),
            pl.BlockSpec((BM2, 1), lambda i: (i, 0)),
        ],
        out_specs=pl.BlockSpec((BM2, DEPTH, HIDDEN), lambda i: (i, 0, 0)),
        out_shape=jax.ShapeDtypeStruct((B, DEPTH, HIDDEN), jnp.float32),
        compiler_params=pltpu.CompilerParams(
            dimension_semantics=("arbitrary",),
        ),
    )(stack, top_base, wslot)


CHUNK = 16
TRASH = B
B_PAD = B + 8


def _k3_body(stack_hbm, base_hbm, idx_hbm, out_hbm, idx_v, gsafe_v, tgt_v,
             rows_v, sem):
    wid = lax.axis_index("s") * 2 + lax.axis_index("c")
    r0 = wid * RPT
    pltpu.sync_copy(idx_hbm.at[pl.ds(r0, RPT)], idx_v)

    for c in range(RPT // CHUNK):
        rows = r0 + c * CHUNK
        g = idx_v[pl.ds(c * CHUNK, CHUNK)]
        gsafe_v[...] = jnp.maximum(g, 0)
        lane = lax.broadcasted_iota(jnp.int32, (CHUNK,), 0)
        tgt_v[...] = jnp.where(g >= 0, rows + lane, TRASH)
        pltpu.sync_copy(base_hbm.at[pl.ds(rows, CHUNK)],
                        out_hbm.at[pl.ds(rows, CHUNK)])
        pltpu.async_copy(stack_hbm.at[gsafe_v], rows_v, sem).wait()
        pltpu.async_copy(rows_v, out_hbm.at[tgt_v], sem).wait()


def _k3(stack_flat, top_base, src_idx):
    mesh = plsc.VectorSubcoreMesh(core_axis_name="c", subcore_axis_name="s")
    run = pl.kernel(
        _k3_body,
        out_type=jax.ShapeDtypeStruct((B_PAD, HIDDEN), jnp.float32),
        mesh=mesh,
        scratch_types=[
            pltpu.VMEM((RPT,), jnp.int32),
            pltpu.VMEM((CHUNK,), jnp.int32),
            pltpu.VMEM((CHUNK,), jnp.int32),
            pltpu.VMEM((CHUNK, HIDDEN), jnp.float32),
            pltpu.SemaphoreType.DMA,
        ],
    )
    return run(stack_flat, top_base, src_idx)


def kernel(hidden_state, stack, pointer, W_push, b_push, W_gate, b_gate,
           ln_gamma, ln_beta):
    bp = b_push.reshape(1, HIDDEN)
    gam = ln_gamma.reshape(1, HIDDEN)
    bet = ln_beta.reshape(1, HIDDEN)
    bg = b_gate.reshape(1, 1)

    top_base, new_pointer, wslot, src_idx = _k1(
        hidden_state, W_push, bp, gam, bet, W_gate, bg, pointer)

    new_stack = _k2(stack, top_base, wslot)
    top_pad = _k3(stack.reshape(B * DEPTH, HIDDEN), top_base,
                  src_idx.reshape(B))
    return (new_stack, new_pointer, top_pad[:B])


# speedup vs baseline: 1.0320x; 1.0320x over previous
import functools

import jax
import jax.numpy as jnp
from jax import lax
from jax.experimental import pallas as pl
from jax.experimental.pallas import tpu as pltpu
from jax.experimental.pallas import tpu_sc as plsc

B = 1024
HIDDEN = 4096
DEPTH = 8
EPS = 1e-5

BM = 256
BK = 512
BM2 = 64
N_TILES = 32
RPT = B // N_TILES


def _k1_body(h_ref, w_ref, bp_ref, g_ref, be_ref, wg_ref, bg_ref, ptr_ref,
             top_ref, nptr_ref, wslot_ref, sidx_ref, acc_ref, gacc_ref):
    k = pl.program_id(1)
    nk = pl.num_programs(1)

    h_f32 = h_ref[...]
    h_bf = h_f32.astype(jnp.bfloat16)
    w_bf = w_ref[...].astype(jnp.bfloat16)
    part = jnp.dot(h_bf, w_bf, preferred_element_type=jnp.float32)
    wg_bf = wg_ref[...].astype(jnp.bfloat16)
    gpart = jnp.dot(h_bf, wg_bf, preferred_element_type=jnp.float32)

    @pl.when(k == 0)
    def _():
        acc_ref[...] = part
        gacc_ref[...] = gpart

    @pl.when(k != 0)
    def _():
        acc_ref[...] += part
        gacc_ref[...] += gpart

    @pl.when(k == nk - 1)
    def _():
        push = acc_ref[...] + bp_ref[...]
        mean = jnp.mean(push, axis=1, keepdims=True)
        cent = push - mean
        var = jnp.mean(cent * cent, axis=1, keepdims=True)
        normed = cent / jnp.sqrt(var + EPS) * g_ref[...] + be_ref[...]

        logit = gacc_ref[...] + bg_ref[...]
        is_pop = logit > 0.0
        ptr = ptr_ref[...].astype(jnp.int32)
        can_pop = is_pop & (ptr > 0)
        can_push = jnp.logical_not(is_pop) & (ptr < DEPTH)

        m = pl.program_id(0)
        row = m * BM + lax.broadcasted_iota(jnp.int32, (BM, 1), 0)

        top_ref[...] = jnp.where(can_push, normed, 0.0)
        nptr_ref[...] = jnp.where(
            can_pop, ptr - 1, jnp.where(can_push, ptr + 1, ptr)
        ).astype(jnp.float32)
        wslot_ref[...] = jnp.where(can_push, ptr, -1)
        gidx = DEPTH * row + jnp.clip(ptr - 1, 0, DEPTH - 1)
        sidx_ref[...] = jnp.where(jnp.logical_not(can_push) & (ptr > 0),
                                  gidx, -1)


def _k1(hidden_state, w_push, b_push, ln_gamma, ln_beta, w_gate, b_gate,
        pointer):
    nk = HIDDEN // BK
    grid = (B // BM, nk)
    return pl.pallas_call(
        _k1_body,
        grid=grid,
        in_specs=[
            pl.BlockSpec((BM, BK), lambda m, k: (m, k)),
            pl.BlockSpec((BK, HIDDEN), lambda m, k: (k, 0)),
            pl.BlockSpec((1, HIDDEN), lambda m, k: (0, 0)),
            pl.BlockSpec((1, HIDDEN), lambda m, k: (0, 0)),
            pl.BlockSpec((1, HIDDEN), lambda m, k: (0, 0)),
            pl.BlockSpec((BK, 1), lambda m, k: (k, 0)),
            pl.BlockSpec((1, 1), lambda m, k: (0, 0)),
            pl.BlockSpec((BM, 1), lambda m, k: (m, 0)),
        ],
        out_specs=[
            pl.BlockSpec((BM, HIDDEN), lambda m, k: (m, 0)),
            pl.BlockSpec((BM, 1), lambda m, k: (m, 0)),
            pl.BlockSpec((BM, 1), lambda m, k: (m, 0)),
            pl.BlockSpec((BM, 1), lambda m, k: (m, 0)),
        ],
        out_shape=[
            jax.ShapeDtypeStruct((B, HIDDEN), jnp.float32),
            jax.ShapeDtypeStruct((B, 1), jnp.float32),
            jax.ShapeDtypeStruct((B, 1), jnp.int32),
            jax.ShapeDtypeStruct((B, 1), jnp.int32),
        ],
        scratch_shapes=[
            pltpu.VMEM((BM, HIDDEN), jnp.float32),
            pltpu.VMEM((BM, 1), jnp.float32),
        ],
        compiler_params=pltpu.CompilerParams(
            dimension_semantics=("parallel", "arbitrary"),
        ),
    )(hidden_state, w_push, b_push, ln_gamma, ln_beta, w_gate, b_gate,
      pointer)


def _k2_body(stack_ref, top_ref, wslot_ref, out_ref):
    ws = wslot_ref[...].reshape(BM2, 1, 1)
    slot = lax.broadcasted_iota(jnp.int32, (BM2, DEPTH, 1), 1)
    push3 = top_ref[...].reshape(BM2, 1, HIDDEN)
    out_ref[...] = jnp.where(slot == ws, push3, stack_ref[...])


def _k2(stack, top_base, wslot):
    grid = (B // BM2,)
    return pl.pallas_call(
        _k2_body,
        grid=grid,
        in_specs=[
            pl.BlockSpec((BM2, DEPTH, HIDDEN), lambda i: (i, 0, 0)),
            pl.BlockSpec((BM2, HIDDEN), lambda i: (i, 0)),
            pl.BlockSpec((BM2, 1), lambda i: (i, 0)),
        ],
        out_specs=pl.BlockSpec((BM2, DEPTH, HIDDEN), lambda i: (i, 0, 0)),
        out_shape=jax.ShapeDtypeStruct((B, DEPTH, HIDDEN), jnp.float32),
        compiler_params=pltpu.CompilerParams(
            dimension_semantics=("arbitrary",),
        ),
    )(stack, top_base, wslot)


N_SCS = 2
RPS = B // N_SCS


def _k3_body(stack_hbm, base_hbm, idx_hbm, out_hbm, idx_s, sem_i, sem):
    core = lax.axis_index("c")
    r0 = core * RPS
    pltpu.async_copy(idx_hbm.at[pl.ds(r0, RPS)], idx_s, sem_i).wait()

    def issue(i, carry):
        g = idx_s[i]

        @pl.when(g >= 0)
        def _():
            pltpu.make_async_copy(
                stack_hbm.at[pl.ds(g, 1)], out_hbm.at[pl.ds(r0 + i, 1)], sem
            ).start()

        @pl.when(g < 0)
        def _():
            pltpu.make_async_copy(
                base_hbm.at[pl.ds(r0 + i, 1)], out_hbm.at[pl.ds(r0 + i, 1)],
                sem,
            ).start()

        return carry

    lax.fori_loop(0, RPS, issue, 0, unroll=4)

    def drain(i, carry):
        pltpu.make_async_copy(
            base_hbm.at[pl.ds(r0, 1)], out_hbm.at[pl.ds(r0, 1)], sem
        ).wait()
        return carry

    lax.fori_loop(0, RPS, drain, 0, unroll=4)


def _k3(stack_flat, top_base, src_idx):
    mesh = plsc.ScalarSubcoreMesh(axis_name="c", num_cores=N_SCS)
    run = pl.kernel(
        _k3_body,
        out_type=jax.ShapeDtypeStruct((B, HIDDEN), jnp.float32),
        mesh=mesh,
        scratch_types=[
            pltpu.SMEM((RPS,), jnp.int32),
            pltpu.SemaphoreType.DMA,
            pltpu.SemaphoreType.DMA,
        ],
    )
    return run(stack_flat, top_base, src_idx)


def kernel(hidden_state, stack, pointer, W_push, b_push, W_gate, b_gate,
           ln_gamma, ln_beta):
    bp = b_push.reshape(1, HIDDEN)
    gam = ln_gamma.reshape(1, HIDDEN)
    bet = ln_beta.reshape(1, HIDDEN)
    bg = b_gate.reshape(1, 1)

    top_base, new_pointer, wslot, src_idx = _k1(
        hidden_state, W_push, bp, gam, bet, W_gate, bg, pointer)

    new_stack = _k2(stack, top_base, wslot)
    top_item = _k3(stack.reshape(B * DEPTH, HIDDEN), top_base,
                   src_idx.reshape(B))
    return (new_stack, new_pointer, top_item)
